# baseline (speedup 1.0000x reference)
import functools

import jax
import jax.numpy as jnp
from jax import lax
from jax.experimental import pallas as pl
from jax.experimental.pallas import tpu as pltpu
from jax.experimental.pallas import tpu_sc as plsc

N = 10000
E = 160000
D_IN = 1433
D_HID = 16
D_OUT = 7

NC, NS = 2, 16
NW = NC * NS
CHUNK = 1280
CPT = 4
E_PAD = NW * CPT * CHUNK
TRASH = N
ACC_ROWS = 10240
RPT = 624
REM = N - NS * RPT
REM_BASE = NS * RPT

_sc_mesh = plsc.VectorSubcoreMesh(core_axis_name="c", subcore_axis_name="s")


@functools.partial(
    pl.kernel,
    out_type=jax.ShapeDtypeStruct((NC, N, D_HID), jnp.float32),
    mesh=_sc_mesh,
    scratch_types=[
        pltpu.VMEM((CPT, CHUNK), jnp.int32),
        pltpu.VMEM((CPT, CHUNK), jnp.int32),
        pltpu.VMEM((CPT, CHUNK, D_HID), jnp.float32),
        pltpu.VMEM_SHARED((ACC_ROWS, D_HID), jnp.float32),
        pltpu.VMEM_SHARED((ACC_ROWS, D_HID), jnp.float32),
        pltpu.SemaphoreType.DMA,
    ],
    compiler_params=pltpu.CompilerParams(use_tc_tiling_on_sc=False),
)
def _sc_aggregate(p_hbm, sidx_hbm, didx_hbm, out_hbm, sidx_v, didx_v, rows_v, acc_sh, stage_sh, gsem):
    c = lax.axis_index("c")
    s = lax.axis_index("s")
    wid = c * NS + s

    pltpu.sync_copy(sidx_hbm.at[wid], sidx_v)
    pltpu.sync_copy(didx_hbm.at[wid], didx_v)

    pltpu.sync_copy(p_hbm.at[pl.ds(s * RPT, RPT)], stage_sh.at[pl.ds(s * RPT, RPT)])
    pltpu.sync_copy(p_hbm.at[pl.ds(s * RPT, RPT)], acc_sh.at[pl.ds(s * RPT, RPT)])

    @pl.when(s == 0)
    def _():
        pltpu.sync_copy(p_hbm.at[pl.ds(REM_BASE, REM)], stage_sh.at[pl.ds(REM_BASE, REM)])
        pltpu.sync_copy(p_hbm.at[pl.ds(REM_BASE, REM)], acc_sh.at[pl.ds(REM_BASE, REM)])
    plsc.subcore_barrier()

    def fire(j, carry):
        pltpu.make_async_copy(stage_sh.at[sidx_v.at[j]], rows_v.at[j], gsem).start()
        return carry
    lax.fori_loop(0, CPT, fire, 0)

    def drain(j, carry):
        pltpu.make_async_copy(stage_sh.at[sidx_v.at[j]], rows_v.at[j], gsem).wait()
        return carry
    lax.fori_loop(0, CPT, drain, 0)

    def scat(j, carry):
        pltpu.async_copy(rows_v.at[j], acc_sh.at[didx_v.at[j]], gsem, add=True)
        return carry
    lax.fori_loop(0, CPT, scat, 0)

    def sdrain(j, carry):
        pltpu.make_async_copy(rows_v.at[j], acc_sh.at[didx_v.at[j]], gsem).wait()
        return carry
    lax.fori_loop(0, CPT, sdrain, 0)
    plsc.subcore_barrier()

    pltpu.sync_copy(acc_sh.at[pl.ds(s * RPT, RPT)], out_hbm.at[c, pl.ds(s * RPT, RPT)])

    @pl.when(s == 0)
    def _():
        pltpu.sync_copy(acc_sh.at[pl.ds(REM_BASE, REM)], out_hbm.at[c, pl.ds(REM_BASE, REM)])


def _mm1_body(x_ref, w_ref, o_ref):
    o_ref[...] = jnp.dot(x_ref[...], w_ref[...],
                         preferred_element_type=jnp.float32)[:, :D_HID]


def _combine_relu_body(a_ref, p_ref, b_ref, o_ref):
    o_ref[...] = jnp.maximum(a_ref[0] + a_ref[1] - p_ref[...] + b_ref[...], 0.0)


def _mm2_body(q_ref, h_ref, w_ref, b_ref, o_ref):
    agg = q_ref[0] + q_ref[1] - h_ref[...]
    o_ref[...] = jnp.dot(agg, w_ref[...], preferred_element_type=jnp.float32) + b_ref[...]


def kernel(features, edge_index, W1, b1, W2, b2):
    src = edge_index[0]
    dst = edge_index[1]
    pad = E_PAD - E
    src_p = jnp.concatenate([src, jnp.zeros((pad,), jnp.int32)]).reshape(NW, CPT, CHUNK)
    dst_p = jnp.concatenate([dst, jnp.full((pad,), TRASH, jnp.int32)]).reshape(NW, CPT, CHUNK)

    BM = 2000
    W1p = jnp.pad(W1, ((0, 0), (0, 128 - D_HID)))
    p1 = pl.pallas_call(
        _mm1_body,
        grid=(N // BM,),
        in_specs=[
            pl.BlockSpec((BM, D_IN), lambda i: (i, 0)),
            pl.BlockSpec((D_IN, 128), lambda i: (0, 0)),
        ],
        out_specs=pl.BlockSpec((BM, D_HID), lambda i: (i, 0)),
        out_shape=jax.ShapeDtypeStruct((N, D_HID), jnp.float32),
    )(features, W1p)

    parts1 = _sc_aggregate(p1, src_p, dst_p)

    BC = 2000
    h1 = pl.pallas_call(
        _combine_relu_body,
        grid=(N // BC,),
        in_specs=[
            pl.BlockSpec((NC, BC, D_HID), lambda i: (0, i, 0)),
            pl.BlockSpec((BC, D_HID), lambda i: (i, 0)),
            pl.BlockSpec((1, D_HID), lambda i: (0, 0)),
        ],
        out_specs=pl.BlockSpec((BC, D_HID), lambda i: (i, 0)),
        out_shape=jax.ShapeDtypeStruct((N, D_HID), jnp.float32),
    )(parts1, p1, b1.reshape(1, D_HID))

    parts2 = _sc_aggregate(h1, src_p, dst_p)

    W2p = jnp.pad(W2, ((0, 0), (0, 8 - D_OUT)))
    b2p = jnp.pad(b2, (0, 8 - D_OUT)).reshape(1, 8)
    out8 = pl.pallas_call(
        _mm2_body,
        grid=(N // BC,),
        in_specs=[
            pl.BlockSpec((NC, BC, D_HID), lambda i: (0, i, 0)),
            pl.BlockSpec((BC, D_HID), lambda i: (i, 0)),
            pl.BlockSpec((D_HID, 8), lambda i: (0, 0)),
            pl.BlockSpec((1, 8), lambda i: (0, 0)),
        ],
        out_specs=pl.BlockSpec((BC, 8), lambda i: (i, 0)),
        out_shape=jax.ShapeDtypeStruct((N, 8), jnp.float32),
    )(parts2, h1, W2p, b2p)

    return out8[:, :D_OUT]


# speedup vs baseline: 1.0198x; 1.0198x over previous
import functools

import jax
import jax.numpy as jnp
from jax import lax
from jax.experimental import pallas as pl
from jax.experimental.pallas import tpu as pltpu
from jax.experimental.pallas import tpu_sc as plsc

N = 10000
E = 160000
D_IN = 1433
D_HID = 16
D_OUT = 7

NC, NS = 2, 16
NW = NC * NS
CHUNK = 1280
CPT = 4
E_PAD = NW * CPT * CHUNK
TRASH = N
ACC_ROWS = 10240
RPT = 624
REM = N - NS * RPT
REM_BASE = NS * RPT

_sc_mesh = plsc.VectorSubcoreMesh(core_axis_name="c", subcore_axis_name="s")


@functools.partial(
    pl.kernel,
    out_type=jax.ShapeDtypeStruct((NC, N, D_HID), jnp.float32),
    mesh=_sc_mesh,
    scratch_types=[
        pltpu.VMEM((CPT, CHUNK), jnp.int32),
        pltpu.VMEM((CPT, CHUNK), jnp.int32),
        pltpu.VMEM((CPT, CHUNK, D_HID), jnp.float32),
        pltpu.VMEM_SHARED((ACC_ROWS, D_HID), jnp.float32),
        pltpu.VMEM_SHARED((ACC_ROWS, D_HID), jnp.float32),
        pltpu.SemaphoreType.DMA,
    ],
    compiler_params=pltpu.CompilerParams(use_tc_tiling_on_sc=False),
)
def _sc_aggregate(p_hbm, sidx_hbm, didx_hbm, out_hbm, sidx_v, didx_v, rows_v, acc_sh, stage_sh, gsem):
    c = lax.axis_index("c")
    s = lax.axis_index("s")
    wid = c * NS + s

    pltpu.sync_copy(sidx_hbm.at[wid], sidx_v)
    pltpu.sync_copy(didx_hbm.at[wid], didx_v)

    pltpu.sync_copy(p_hbm.at[pl.ds(s * RPT, RPT)], stage_sh.at[pl.ds(s * RPT, RPT)])
    pltpu.sync_copy(p_hbm.at[pl.ds(s * RPT, RPT)], acc_sh.at[pl.ds(s * RPT, RPT)])

    @pl.when(s == 0)
    def _():
        pltpu.sync_copy(p_hbm.at[pl.ds(REM_BASE, REM)], stage_sh.at[pl.ds(REM_BASE, REM)])
        pltpu.sync_copy(p_hbm.at[pl.ds(REM_BASE, REM)], acc_sh.at[pl.ds(REM_BASE, REM)])
    plsc.subcore_barrier()

    def fire(j, carry):
        pltpu.make_async_copy(stage_sh.at[sidx_v.at[j]], rows_v.at[j], gsem).start()
        return carry
    lax.fori_loop(0, CPT, fire, 0)

    def drain(j, carry):
        pltpu.make_async_copy(stage_sh.at[sidx_v.at[j]], rows_v.at[j], gsem).wait()
        return carry
    lax.fori_loop(0, CPT, drain, 0)

    def scat(j, carry):
        pltpu.async_copy(rows_v.at[j], acc_sh.at[didx_v.at[j]], gsem, add=True)
        return carry
    lax.fori_loop(0, CPT, scat, 0)

    def sdrain(j, carry):
        pltpu.make_async_copy(rows_v.at[j], acc_sh.at[didx_v.at[j]], gsem).wait()
        return carry
    lax.fori_loop(0, CPT, sdrain, 0)
    plsc.subcore_barrier()

    pltpu.sync_copy(acc_sh.at[pl.ds(s * RPT, RPT)], out_hbm.at[c, pl.ds(s * RPT, RPT)])

    @pl.when(s == 0)
    def _():
        pltpu.sync_copy(acc_sh.at[pl.ds(REM_BASE, REM)], out_hbm.at[c, pl.ds(REM_BASE, REM)])


@functools.partial(
    pl.kernel,
    out_type=(
        jax.ShapeDtypeStruct((N, D_HID), jnp.float32),
        jax.ShapeDtypeStruct((NC, N, D_HID), jnp.float32),
    ),
    mesh=_sc_mesh,
    scratch_types=[
        pltpu.VMEM((CPT, CHUNK), jnp.int32),
        pltpu.VMEM((CPT, CHUNK), jnp.int32),
        pltpu.VMEM((2, CHUNK, D_HID), jnp.float32),
        pltpu.VMEM((RPT, D_HID), jnp.float32),
        pltpu.VMEM((RPT, D_HID), jnp.float32),
        pltpu.VMEM((REM, D_HID), jnp.float32),
        pltpu.VMEM((REM, D_HID), jnp.float32),
        pltpu.VMEM((1, D_HID), jnp.float32),
        pltpu.VMEM_SHARED((ACC_ROWS, D_HID), jnp.float32),
        pltpu.VMEM_SHARED((ACC_ROWS, D_HID), jnp.float32),
        pltpu.SemaphoreType.DMA,
    ],
    compiler_params=pltpu.CompilerParams(use_tc_tiling_on_sc=False),
)
def _sc_combine_aggregate(parts_hbm, p_hbm, b_hbm, sidx_hbm, didx_hbm,
                          h_hbm, out_hbm,
                          sidx_v, didx_v, rows_v, hbuf, tbuf, hrem, trem,
                          bbuf, acc_sh, stage_sh, gsem):
    c = lax.axis_index("c")
    s = lax.axis_index("s")
    wid = c * NS + s

    pltpu.sync_copy(sidx_hbm.at[wid], sidx_v)
    pltpu.sync_copy(didx_hbm.at[wid], didx_v)
    pltpu.sync_copy(b_hbm, bbuf)
    bvec = bbuf[0]

    def compute_h(base, nrows, hb, tb):
        pltpu.sync_copy(parts_hbm.at[0, pl.ds(base, nrows)], hb)
        pltpu.sync_copy(parts_hbm.at[1, pl.ds(base, nrows)], tb)

        def add1(i, carry):
            hb[i] = hb[i] + tb[i]
            return carry
        lax.fori_loop(0, nrows, add1, 0)
        pltpu.sync_copy(p_hbm.at[pl.ds(base, nrows)], tb)

        def relu1(i, carry):
            hb[i] = jnp.maximum(hb[i] - tb[i] + bvec, 0.0)
            return carry
        lax.fori_loop(0, nrows, relu1, 0)
        pltpu.sync_copy(hb, stage_sh.at[pl.ds(base, nrows)])
        pltpu.sync_copy(hb, acc_sh.at[pl.ds(base, nrows)])

        @pl.when(c == 0)
        def _():
            pltpu.sync_copy(hb, h_hbm.at[pl.ds(base, nrows)])

    compute_h(s * RPT, RPT, hbuf, tbuf)

    @pl.when(s == 0)
    def _():
        compute_h(REM_BASE, REM, hrem, trem)
    plsc.subcore_barrier()

    def agg_pass(p, carry):
        for k in range(2):
            pltpu.make_async_copy(
                stage_sh.at[sidx_v.at[2 * p + k]], rows_v.at[k], gsem).start()
        for k in range(2):
            pltpu.make_async_copy(
                stage_sh.at[sidx_v.at[2 * p + k]], rows_v.at[k], gsem).wait()
        for k in range(2):
            pltpu.async_copy(
                rows_v.at[k], acc_sh.at[didx_v.at[2 * p + k]], gsem, add=True)
        for k in range(2):
            pltpu.make_async_copy(
                rows_v.at[k], acc_sh.at[didx_v.at[2 * p + k]], gsem).wait()
        return carry
    lax.fori_loop(0, CPT // 2, agg_pass, 0)
    plsc.subcore_barrier()

    pltpu.sync_copy(acc_sh.at[pl.ds(s * RPT, RPT)], out_hbm.at[c, pl.ds(s * RPT, RPT)])

    @pl.when(s == 0)
    def _():
        pltpu.sync_copy(acc_sh.at[pl.ds(REM_BASE, REM)], out_hbm.at[c, pl.ds(REM_BASE, REM)])


def _mm1_body(x_ref, w_ref, o_ref):
    o_ref[...] = jnp.dot(x_ref[...], w_ref[...],
                         preferred_element_type=jnp.float32)[:, :D_HID]


def _combine_relu_body(a_ref, p_ref, b_ref, o_ref):
    o_ref[...] = jnp.maximum(a_ref[0] + a_ref[1] - p_ref[...] + b_ref[...], 0.0)


def _mm2_body(q_ref, h_ref, w_ref, b_ref, o_ref):
    agg = q_ref[0] + q_ref[1] - h_ref[...]
    o_ref[...] = jnp.dot(agg, w_ref[...], preferred_element_type=jnp.float32) + b_ref[...]


def kernel(features, edge_index, W1, b1, W2, b2):
    src = edge_index[0]
    dst = edge_index[1]
    pad = E_PAD - E
    src_p = jnp.concatenate([src, jnp.zeros((pad,), jnp.int32)]).reshape(NW, CPT, CHUNK)
    dst_p = jnp.concatenate([dst, jnp.full((pad,), TRASH, jnp.int32)]).reshape(NW, CPT, CHUNK)

    BM = 2000
    W1p = jnp.pad(W1, ((0, 0), (0, 128 - D_HID)))
    p1 = pl.pallas_call(
        _mm1_body,
        grid=(N // BM,),
        in_specs=[
            pl.BlockSpec((BM, D_IN), lambda i: (i, 0)),
            pl.BlockSpec((D_IN, 128), lambda i: (0, 0)),
        ],
        out_specs=pl.BlockSpec((BM, D_HID), lambda i: (i, 0)),
        out_shape=jax.ShapeDtypeStruct((N, D_HID), jnp.float32),
    )(features, W1p)

    parts1 = _sc_aggregate(p1, src_p, dst_p)

    BC = 2000
    h1, parts2 = _sc_combine_aggregate(parts1, p1, b1.reshape(1, D_HID), src_p, dst_p)

    W2p = jnp.pad(W2, ((0, 0), (0, 8 - D_OUT)))
    b2p = jnp.pad(b2, (0, 8 - D_OUT)).reshape(1, 8)
    out8 = pl.pallas_call(
        _mm2_body,
        grid=(N // BC,),
        in_specs=[
            pl.BlockSpec((NC, BC, D_HID), lambda i: (0, i, 0)),
            pl.BlockSpec((BC, D_HID), lambda i: (i, 0)),
            pl.BlockSpec((D_HID, 8), lambda i: (0, 0)),
            pl.BlockSpec((1, 8), lambda i: (0, 0)),
        ],
        out_specs=pl.BlockSpec((BC, 8), lambda i: (i, 0)),
        out_shape=jax.ShapeDtypeStruct((N, 8), jnp.float32),
    )(parts2, h1, W2p, b2p)

    return out8[:, :D_OUT]


# speedup vs baseline: 1.0200x; 1.0003x over previous
import functools

import jax
import jax.numpy as jnp
from jax import lax
from jax.experimental import pallas as pl
from jax.experimental.pallas import tpu as pltpu
from jax.experimental.pallas import tpu_sc as plsc

N = 10000
E = 160000
D_IN = 1433
D_HID = 16
D_OUT = 7

NC, NS = 2, 16
NW = NC * NS
CHUNK = 1280
CPT = 4
E_PAD = NW * CPT * CHUNK
TRASH = N
ACC_ROWS = 10240
RPT = 624
REM = N - NS * RPT
REM_BASE = NS * RPT

_sc_mesh = plsc.VectorSubcoreMesh(core_axis_name="c", subcore_axis_name="s")


@functools.partial(
    pl.kernel,
    out_type=jax.ShapeDtypeStruct((NC, N, D_HID), jnp.float32),
    mesh=_sc_mesh,
    scratch_types=[
        pltpu.VMEM((CPT, CHUNK), jnp.int32),
        pltpu.VMEM((CPT, CHUNK), jnp.int32),
        pltpu.VMEM((CPT, CHUNK, D_HID), jnp.float32),
        pltpu.VMEM_SHARED((ACC_ROWS, D_HID), jnp.float32),
        pltpu.VMEM_SHARED((ACC_ROWS, D_HID), jnp.float32),
        pltpu.SemaphoreType.DMA,
    ],
    compiler_params=pltpu.CompilerParams(use_tc_tiling_on_sc=False),
)
def _sc_aggregate(p_hbm, sidx_hbm, didx_hbm, out_hbm, sidx_v, didx_v, rows_v, acc_sh, stage_sh, gsem):
    c = lax.axis_index("c")
    s = lax.axis_index("s")
    wid = c * NS + s

    pltpu.sync_copy(sidx_hbm.at[wid], sidx_v)
    pltpu.sync_copy(didx_hbm.at[wid], didx_v)

    pltpu.sync_copy(p_hbm.at[pl.ds(s * RPT, RPT)], stage_sh.at[pl.ds(s * RPT, RPT)])
    pltpu.sync_copy(p_hbm.at[pl.ds(s * RPT, RPT)], acc_sh.at[pl.ds(s * RPT, RPT)])

    @pl.when(s == 0)
    def _():
        pltpu.sync_copy(p_hbm.at[pl.ds(REM_BASE, REM)], stage_sh.at[pl.ds(REM_BASE, REM)])
        pltpu.sync_copy(p_hbm.at[pl.ds(REM_BASE, REM)], acc_sh.at[pl.ds(REM_BASE, REM)])
    plsc.subcore_barrier()

    def fire(j, carry):
        pltpu.make_async_copy(stage_sh.at[sidx_v.at[j]], rows_v.at[j], gsem).start()
        return carry
    lax.fori_loop(0, CPT, fire, 0)

    def drain(j, carry):
        pltpu.make_async_copy(stage_sh.at[sidx_v.at[j]], rows_v.at[j], gsem).wait()
        return carry
    lax.fori_loop(0, CPT, drain, 0)

    def scat(j, carry):
        pltpu.async_copy(rows_v.at[j], acc_sh.at[didx_v.at[j]], gsem, add=True)
        return carry
    lax.fori_loop(0, CPT, scat, 0)

    def sdrain(j, carry):
        pltpu.make_async_copy(rows_v.at[j], acc_sh.at[didx_v.at[j]], gsem).wait()
        return carry
    lax.fori_loop(0, CPT, sdrain, 0)
    plsc.subcore_barrier()

    pltpu.sync_copy(acc_sh.at[pl.ds(s * RPT, RPT)], out_hbm.at[c, pl.ds(s * RPT, RPT)])

    @pl.when(s == 0)
    def _():
        pltpu.sync_copy(acc_sh.at[pl.ds(REM_BASE, REM)], out_hbm.at[c, pl.ds(REM_BASE, REM)])


@functools.partial(
    pl.kernel,
    out_type=(
        jax.ShapeDtypeStruct((N, D_HID), jnp.float32),
        jax.ShapeDtypeStruct((NC, N, D_HID), jnp.float32),
    ),
    mesh=_sc_mesh,
    scratch_types=[
        pltpu.VMEM((CPT, CHUNK), jnp.int32),
        pltpu.VMEM((CPT, CHUNK), jnp.int32),
        pltpu.VMEM((2, CHUNK, D_HID), jnp.float32),
        pltpu.VMEM((RPT, D_HID), jnp.float32),
        pltpu.VMEM((RPT, D_HID), jnp.float32),
        pltpu.VMEM((REM, D_HID), jnp.float32),
        pltpu.VMEM((REM, D_HID), jnp.float32),
        pltpu.VMEM((1, D_HID), jnp.float32),
        pltpu.VMEM_SHARED((ACC_ROWS, D_HID), jnp.float32),
        pltpu.VMEM_SHARED((ACC_ROWS, D_HID), jnp.float32),
        pltpu.SemaphoreType.DMA,
    ],
    compiler_params=pltpu.CompilerParams(use_tc_tiling_on_sc=False),
)
def _sc_combine_aggregate(parts_hbm, p_hbm, b_hbm, sidx_hbm, didx_hbm,
                          h_hbm, out_hbm,
                          sidx_v, didx_v, rows_v, hbuf, tbuf, hrem, trem,
                          bbuf, acc_sh, stage_sh, gsem):
    c = lax.axis_index("c")
    s = lax.axis_index("s")
    wid = c * NS + s

    pltpu.sync_copy(sidx_hbm.at[wid], sidx_v)
    pltpu.sync_copy(didx_hbm.at[wid], didx_v)
    pltpu.sync_copy(b_hbm, bbuf)
    bvec = bbuf[0]

    def compute_h(base, nrows, hb, tb):
        pltpu.sync_copy(parts_hbm.at[0, pl.ds(base, nrows)], hb)
        pltpu.sync_copy(parts_hbm.at[1, pl.ds(base, nrows)], tb)

        def add1(i, carry):
            hb[i] = hb[i] + tb[i]
            return carry
        lax.fori_loop(0, nrows, add1, 0)
        pltpu.sync_copy(p_hbm.at[pl.ds(base, nrows)], tb)

        def relu1(i, carry):
            hb[i] = jnp.maximum(hb[i] - tb[i] + bvec, 0.0)
            return carry
        lax.fori_loop(0, nrows, relu1, 0)
        pltpu.sync_copy(hb, stage_sh.at[pl.ds(base, nrows)])
        pltpu.sync_copy(hb, acc_sh.at[pl.ds(base, nrows)])

        @pl.when(c == 0)
        def _():
            pltpu.sync_copy(hb, h_hbm.at[pl.ds(base, nrows)])

    compute_h(s * RPT, RPT, hbuf, tbuf)

    @pl.when(s == 0)
    def _():
        compute_h(REM_BASE, REM, hrem, trem)
    plsc.subcore_barrier()

    def agg_pass(p, carry):
        for k in range(2):
            pltpu.make_async_copy(
                stage_sh.at[sidx_v.at[2 * p + k]], rows_v.at[k], gsem).start()
        for k in range(2):
            pltpu.make_async_copy(
                stage_sh.at[sidx_v.at[2 * p + k]], rows_v.at[k], gsem).wait()
        for k in range(2):
            pltpu.async_copy(
                rows_v.at[k], acc_sh.at[didx_v.at[2 * p + k]], gsem, add=True)
        for k in range(2):
            pltpu.make_async_copy(
                rows_v.at[k], acc_sh.at[didx_v.at[2 * p + k]], gsem).wait()
        return carry
    lax.fori_loop(0, CPT // 2, agg_pass, 0)
    plsc.subcore_barrier()

    pltpu.sync_copy(acc_sh.at[pl.ds(s * RPT, RPT)], out_hbm.at[c, pl.ds(s * RPT, RPT)])

    @pl.when(s == 0)
    def _():
        pltpu.sync_copy(acc_sh.at[pl.ds(REM_BASE, REM)], out_hbm.at[c, pl.ds(REM_BASE, REM)])


def _mm1_body(x_ref, w_ref, o_ref):
    o_ref[...] = jnp.dot(x_ref[...], w_ref[...],
                         preferred_element_type=jnp.float32)[:, :D_HID]


def _mm2_body(q_ref, h_ref, w_ref, b_ref, o_ref):
    agg = q_ref[0] + q_ref[1] - h_ref[...]
    o_ref[...] = jnp.dot(agg, w_ref[...], preferred_element_type=jnp.float32) + b_ref[...]


def kernel(features, edge_index, W1, b1, W2, b2):
    src = edge_index[0]
    dst = edge_index[1]
    pad = E_PAD - E
    src_p = jnp.concatenate([src, jnp.zeros((pad,), jnp.int32)]).reshape(NW, CPT, CHUNK)
    dst_p = jnp.concatenate([dst, jnp.full((pad,), TRASH, jnp.int32)]).reshape(NW, CPT, CHUNK)

    BM = 2000
    W1p = jnp.pad(W1, ((0, 0), (0, 128 - D_HID)))
    p1 = pl.pallas_call(
        _mm1_body,
        grid=(N // BM,),
        in_specs=[
            pl.BlockSpec((BM, D_IN), lambda i: (i, 0)),
            pl.BlockSpec((D_IN, 128), lambda i: (0, 0)),
        ],
        out_specs=pl.BlockSpec((BM, D_HID), lambda i: (i, 0)),
        out_shape=jax.ShapeDtypeStruct((N, D_HID), jnp.float32),
    )(features, W1p)

    parts1 = _sc_aggregate(p1, src_p, dst_p)

    BC = 2000
    h1, parts2 = _sc_combine_aggregate(parts1, p1, b1.reshape(1, D_HID), src_p, dst_p)

    W2p = jnp.pad(W2, ((0, 0), (0, 8 - D_OUT)))
    b2p = jnp.pad(b2, (0, 8 - D_OUT)).reshape(1, 8)
    out8 = pl.pallas_call(
        _mm2_body,
        grid=(N // BC,),
        in_specs=[
            pl.BlockSpec((NC, BC, D_HID), lambda i: (0, i, 0)),
            pl.BlockSpec((BC, D_HID), lambda i: (i, 0)),
            pl.BlockSpec((D_HID, 8), lambda i: (0, 0)),
            pl.BlockSpec((1, 8), lambda i: (0, 0)),
        ],
        out_specs=pl.BlockSpec((BC, 8), lambda i: (i, 0)),
        out_shape=jax.ShapeDtypeStruct((N, 8), jnp.float32),
    )(parts2, h1, W2p, b2p)

    return out8[:, :D_OUT]


# speedup vs baseline: 1.0360x; 1.0157x over previous
import functools

import jax
import jax.numpy as jnp
from jax import lax
from jax.experimental import pallas as pl
from jax.experimental.pallas import tpu as pltpu
from jax.experimental.pallas import tpu_sc as plsc

N = 10000
E = 160000
D_IN = 1433
D_HID = 16
D_OUT = 7

NC, NS = 2, 16
NW = NC * NS
CHUNK = 1280
CPT = 4
E_PAD = NW * CPT * CHUNK
TRASH = N
ACC_ROWS = 10240
RPT = 624
REM = N - NS * RPT
REM_BASE = NS * RPT

_sc_mesh = plsc.VectorSubcoreMesh(core_axis_name="c", subcore_axis_name="s")


@functools.partial(
    pl.kernel,
    out_type=jax.ShapeDtypeStruct((NC, N, D_HID), jnp.float32),
    mesh=_sc_mesh,
    scratch_types=[
        pltpu.VMEM((CPT, CHUNK), jnp.int32),
        pltpu.VMEM((CPT, CHUNK), jnp.int32),
        pltpu.VMEM((CPT, CHUNK, D_HID), jnp.float32),
        pltpu.VMEM_SHARED((ACC_ROWS, D_HID), jnp.float32),
        pltpu.VMEM_SHARED((ACC_ROWS, D_HID), jnp.float32),
        pltpu.SemaphoreType.DMA,
    ],
    compiler_params=pltpu.CompilerParams(use_tc_tiling_on_sc=False),
)
def _sc_aggregate(p_hbm, sidx_hbm, didx_hbm, out_hbm, sidx_v, didx_v, rows_v, acc_sh, stage_sh, gsem):
    c = lax.axis_index("c")
    s = lax.axis_index("s")
    wid = c * NS + s

    pltpu.make_async_copy(sidx_hbm.at[wid], sidx_v, gsem).start()
    pltpu.make_async_copy(didx_hbm.at[wid], didx_v, gsem).start()
    sl = pl.ds(s * RPT, RPT)
    pltpu.make_async_copy(p_hbm.at[sl], stage_sh.at[sl], gsem).start()
    pltpu.make_async_copy(p_hbm.at[sl], acc_sh.at[sl], gsem).start()

    rl = pl.ds(REM_BASE, REM)

    @pl.when(s == 0)
    def _():
        pltpu.make_async_copy(p_hbm.at[rl], stage_sh.at[rl], gsem).start()
        pltpu.make_async_copy(p_hbm.at[rl], acc_sh.at[rl], gsem).start()

    pltpu.make_async_copy(sidx_hbm.at[wid], sidx_v, gsem).wait()
    pltpu.make_async_copy(didx_hbm.at[wid], didx_v, gsem).wait()
    pltpu.make_async_copy(p_hbm.at[sl], stage_sh.at[sl], gsem).wait()
    pltpu.make_async_copy(p_hbm.at[sl], acc_sh.at[sl], gsem).wait()

    @pl.when(s == 0)
    def _():
        pltpu.make_async_copy(p_hbm.at[rl], stage_sh.at[rl], gsem).wait()
        pltpu.make_async_copy(p_hbm.at[rl], acc_sh.at[rl], gsem).wait()
    plsc.subcore_barrier()

    def fire(j, carry):
        pltpu.make_async_copy(stage_sh.at[sidx_v.at[j]], rows_v.at[j], gsem).start()
        return carry
    lax.fori_loop(0, CPT, fire, 0)

    def drain(j, carry):
        pltpu.make_async_copy(stage_sh.at[sidx_v.at[j]], rows_v.at[j], gsem).wait()
        return carry
    lax.fori_loop(0, CPT, drain, 0)

    def scat(j, carry):
        pltpu.async_copy(rows_v.at[j], acc_sh.at[didx_v.at[j]], gsem, add=True)
        return carry
    lax.fori_loop(0, CPT, scat, 0)

    def sdrain(j, carry):
        pltpu.make_async_copy(rows_v.at[j], acc_sh.at[didx_v.at[j]], gsem).wait()
        return carry
    lax.fori_loop(0, CPT, sdrain, 0)
    plsc.subcore_barrier()

    pltpu.sync_copy(acc_sh.at[pl.ds(s * RPT, RPT)], out_hbm.at[c, pl.ds(s * RPT, RPT)])

    @pl.when(s == 0)
    def _():
        pltpu.sync_copy(acc_sh.at[pl.ds(REM_BASE, REM)], out_hbm.at[c, pl.ds(REM_BASE, REM)])


@functools.partial(
    pl.kernel,
    out_type=(
        jax.ShapeDtypeStruct((N, D_HID), jnp.float32),
        jax.ShapeDtypeStruct((NC, N, D_HID), jnp.float32),
    ),
    mesh=_sc_mesh,
    scratch_types=[
        pltpu.VMEM((CPT, CHUNK), jnp.int32),
        pltpu.VMEM((CPT, CHUNK), jnp.int32),
        pltpu.VMEM((2, CHUNK, D_HID), jnp.float32),
        pltpu.VMEM((RPT, D_HID), jnp.float32),
        pltpu.VMEM((RPT, D_HID), jnp.float32),
        pltpu.VMEM((REM, D_HID), jnp.float32),
        pltpu.VMEM((REM, D_HID), jnp.float32),
        pltpu.VMEM((1, D_HID), jnp.float32),
        pltpu.VMEM_SHARED((ACC_ROWS, D_HID), jnp.float32),
        pltpu.VMEM_SHARED((ACC_ROWS, D_HID), jnp.float32),
        pltpu.SemaphoreType.DMA,
    ],
    compiler_params=pltpu.CompilerParams(use_tc_tiling_on_sc=False),
)
def _sc_combine_aggregate(parts_hbm, p_hbm, b_hbm, sidx_hbm, didx_hbm,
                          h_hbm, out_hbm,
                          sidx_v, didx_v, rows_v, hbuf, tbuf, hrem, trem,
                          bbuf, acc_sh, stage_sh, gsem):
    c = lax.axis_index("c")
    s = lax.axis_index("s")
    wid = c * NS + s

    pltpu.sync_copy(sidx_hbm.at[wid], sidx_v)
    pltpu.sync_copy(didx_hbm.at[wid], didx_v)
    pltpu.sync_copy(b_hbm, bbuf)
    bvec = bbuf[0]

    def compute_h(base, nrows, hb, tb):
        pltpu.sync_copy(parts_hbm.at[0, pl.ds(base, nrows)], hb)
        pltpu.sync_copy(parts_hbm.at[1, pl.ds(base, nrows)], tb)

        def add1(i, carry):
            hb[i] = hb[i] + tb[i]
            return carry
        lax.fori_loop(0, nrows, add1, 0)
        pltpu.sync_copy(p_hbm.at[pl.ds(base, nrows)], tb)

        def relu1(i, carry):
            hb[i] = jnp.maximum(hb[i] - tb[i] + bvec, 0.0)
            return carry
        lax.fori_loop(0, nrows, relu1, 0)
        pltpu.sync_copy(hb, stage_sh.at[pl.ds(base, nrows)])
        pltpu.sync_copy(hb, acc_sh.at[pl.ds(base, nrows)])

        @pl.when(c == 0)
        def _():
            pltpu.sync_copy(hb, h_hbm.at[pl.ds(base, nrows)])

    compute_h(s * RPT, RPT, hbuf, tbuf)

    @pl.when(s == 0)
    def _():
        compute_h(REM_BASE, REM, hrem, trem)
    plsc.subcore_barrier()

    def agg_pass(p, carry):
        for k in range(2):
            pltpu.make_async_copy(
                stage_sh.at[sidx_v.at[2 * p + k]], rows_v.at[k], gsem).start()
        for k in range(2):
            pltpu.make_async_copy(
                stage_sh.at[sidx_v.at[2 * p + k]], rows_v.at[k], gsem).wait()
        for k in range(2):
            pltpu.async_copy(
                rows_v.at[k], acc_sh.at[didx_v.at[2 * p + k]], gsem, add=True)
        for k in range(2):
            pltpu.make_async_copy(
                rows_v.at[k], acc_sh.at[didx_v.at[2 * p + k]], gsem).wait()
        return carry
    lax.fori_loop(0, CPT // 2, agg_pass, 0)
    plsc.subcore_barrier()

    pltpu.sync_copy(acc_sh.at[pl.ds(s * RPT, RPT)], out_hbm.at[c, pl.ds(s * RPT, RPT)])

    @pl.when(s == 0)
    def _():
        pltpu.sync_copy(acc_sh.at[pl.ds(REM_BASE, REM)], out_hbm.at[c, pl.ds(REM_BASE, REM)])


def _mm1_body(x_ref, w_ref, o_ref):
    o_ref[...] = jnp.dot(x_ref[...], w_ref[...],
                         preferred_element_type=jnp.float32)[:, :D_HID]


def _mm2_body(q_ref, h_ref, w_ref, b_ref, o_ref):
    agg = q_ref[0] + q_ref[1] - h_ref[...]
    o_ref[...] = jnp.dot(agg, w_ref[...], preferred_element_type=jnp.float32) + b_ref[...]


def kernel(features, edge_index, W1, b1, W2, b2):
    src = edge_index[0]
    dst = edge_index[1]
    pad = E_PAD - E
    src_p = jnp.concatenate([src, jnp.zeros((pad,), jnp.int32)]).reshape(NW, CPT, CHUNK)
    dst_p = jnp.concatenate([dst, jnp.full((pad,), TRASH, jnp.int32)]).reshape(NW, CPT, CHUNK)

    BM = 2000
    W1p = jnp.pad(W1, ((0, 0), (0, 128 - D_HID)))
    p1 = pl.pallas_call(
        _mm1_body,
        grid=(N // BM,),
        in_specs=[
            pl.BlockSpec((BM, D_IN), lambda i: (i, 0)),
            pl.BlockSpec((D_IN, 128), lambda i: (0, 0)),
        ],
        out_specs=pl.BlockSpec((BM, D_HID), lambda i: (i, 0)),
        out_shape=jax.ShapeDtypeStruct((N, D_HID), jnp.float32),
    )(features, W1p)

    parts1 = _sc_aggregate(p1, src_p, dst_p)

    BC = 2000
    h1, parts2 = _sc_combine_aggregate(parts1, p1, b1.reshape(1, D_HID), src_p, dst_p)

    W2p = jnp.pad(W2, ((0, 0), (0, 8 - D_OUT)))
    b2p = jnp.pad(b2, (0, 8 - D_OUT)).reshape(1, 8)
    out8 = pl.pallas_call(
        _mm2_body,
        grid=(N // BC,),
        in_specs=[
            pl.BlockSpec((NC, BC, D_HID), lambda i: (0, i, 0)),
            pl.BlockSpec((BC, D_HID), lambda i: (i, 0)),
            pl.BlockSpec((D_HID, 8), lambda i: (0, 0)),
            pl.BlockSpec((1, 8), lambda i: (0, 0)),
        ],
        out_specs=pl.BlockSpec((BC, 8), lambda i: (i, 0)),
        out_shape=jax.ShapeDtypeStruct((N, 8), jnp.float32),
    )(parts2, h1, W2p, b2p)

    return out8[:, :D_OUT]


# speedup vs baseline: 1.0752x; 1.0379x over previous
import functools

import jax
import jax.numpy as jnp
from jax import lax
from jax.experimental import pallas as pl
from jax.experimental.pallas import tpu as pltpu
from jax.experimental.pallas import tpu_sc as plsc

N = 10000
E = 160000
D_IN = 1433
D_HID = 16
D_OUT = 7

NC, NS = 2, 16
NW = NC * NS
CHUNK = 1280
CPT = 4
E_PAD = NW * CPT * CHUNK
TRASH = N
ACC_ROWS = 10240
RPT = 624
REM = N - NS * RPT
REM_BASE = NS * RPT

_sc_mesh = plsc.VectorSubcoreMesh(core_axis_name="c", subcore_axis_name="s")


@functools.partial(
    pl.kernel,
    out_type=jax.ShapeDtypeStruct((NC, N, D_HID), jnp.float32),
    mesh=_sc_mesh,
    scratch_types=[
        pltpu.VMEM((CPT, CHUNK), jnp.int32),
        pltpu.VMEM((CPT, CHUNK), jnp.int32),
        pltpu.VMEM((CPT, CHUNK, D_HID), jnp.float32),
        pltpu.VMEM_SHARED((ACC_ROWS, D_HID), jnp.float32),
        pltpu.VMEM_SHARED((ACC_ROWS, D_HID), jnp.float32),
        pltpu.SemaphoreType.DMA,
    ],
    compiler_params=pltpu.CompilerParams(use_tc_tiling_on_sc=False),
)
def _sc_aggregate(p_hbm, sidx_hbm, didx_hbm, out_hbm, sidx_v, didx_v, rows_v, acc_sh, stage_sh, gsem):
    c = lax.axis_index("c")
    s = lax.axis_index("s")
    wid = c * NS + s

    pltpu.make_async_copy(sidx_hbm.at[wid], sidx_v, gsem).start()
    pltpu.make_async_copy(didx_hbm.at[wid], didx_v, gsem).start()
    sl = pl.ds(s * RPT, RPT)
    pltpu.make_async_copy(p_hbm.at[sl], stage_sh.at[sl], gsem).start()
    pltpu.make_async_copy(p_hbm.at[sl], acc_sh.at[sl], gsem).start()

    rl = pl.ds(REM_BASE, REM)

    @pl.when(s == 0)
    def _():
        pltpu.make_async_copy(p_hbm.at[rl], stage_sh.at[rl], gsem).start()
        pltpu.make_async_copy(p_hbm.at[rl], acc_sh.at[rl], gsem).start()

    pltpu.make_async_copy(sidx_hbm.at[wid], sidx_v, gsem).wait()
    pltpu.make_async_copy(didx_hbm.at[wid], didx_v, gsem).wait()
    pltpu.make_async_copy(p_hbm.at[sl], stage_sh.at[sl], gsem).wait()
    pltpu.make_async_copy(p_hbm.at[sl], acc_sh.at[sl], gsem).wait()

    @pl.when(s == 0)
    def _():
        pltpu.make_async_copy(p_hbm.at[rl], stage_sh.at[rl], gsem).wait()
        pltpu.make_async_copy(p_hbm.at[rl], acc_sh.at[rl], gsem).wait()
    plsc.subcore_barrier()

    def fire(j, carry):
        pltpu.make_async_copy(stage_sh.at[sidx_v.at[j]], rows_v.at[j], gsem).start()
        return carry
    lax.fori_loop(0, CPT, fire, 0)

    def drain(j, carry):
        pltpu.make_async_copy(stage_sh.at[sidx_v.at[j]], rows_v.at[j], gsem).wait()
        return carry
    lax.fori_loop(0, CPT, drain, 0)

    def scat(j, carry):
        pltpu.async_copy(rows_v.at[j], acc_sh.at[didx_v.at[j]], gsem, add=True)
        return carry
    lax.fori_loop(0, CPT, scat, 0)

    def sdrain(j, carry):
        pltpu.make_async_copy(rows_v.at[j], acc_sh.at[didx_v.at[j]], gsem).wait()
        return carry
    lax.fori_loop(0, CPT, sdrain, 0)
    plsc.subcore_barrier()

    pltpu.sync_copy(acc_sh.at[pl.ds(s * RPT, RPT)], out_hbm.at[c, pl.ds(s * RPT, RPT)])

    @pl.when(s == 0)
    def _():
        pltpu.sync_copy(acc_sh.at[pl.ds(REM_BASE, REM)], out_hbm.at[c, pl.ds(REM_BASE, REM)])


@functools.partial(
    pl.kernel,
    out_type=(
        jax.ShapeDtypeStruct((N, D_HID), jnp.float32),
        jax.ShapeDtypeStruct((NC, N, D_HID), jnp.float32),
    ),
    mesh=_sc_mesh,
    scratch_types=[
        pltpu.VMEM((CPT, CHUNK), jnp.int32),
        pltpu.VMEM((CPT, CHUNK), jnp.int32),
        pltpu.VMEM((2, CHUNK, D_HID), jnp.float32),
        pltpu.VMEM((RPT, D_HID), jnp.float32),
        pltpu.VMEM((RPT, D_HID), jnp.float32),
        pltpu.VMEM((RPT, D_HID), jnp.float32),
        pltpu.VMEM((REM, D_HID), jnp.float32),
        pltpu.VMEM((REM, D_HID), jnp.float32),
        pltpu.VMEM((REM, D_HID), jnp.float32),
        pltpu.VMEM((1, D_HID), jnp.float32),
        pltpu.VMEM_SHARED((ACC_ROWS, D_HID), jnp.float32),
        pltpu.VMEM_SHARED((ACC_ROWS, D_HID), jnp.float32),
        pltpu.SemaphoreType.DMA,
    ],
    compiler_params=pltpu.CompilerParams(use_tc_tiling_on_sc=False),
)
def _sc_combine_aggregate(parts_hbm, p_hbm, b_hbm, sidx_hbm, didx_hbm,
                          h_hbm, out_hbm,
                          sidx_v, didx_v, rows_v, hbuf, tbuf, pbuf,
                          hrem, trem, prem, bbuf, acc_sh, stage_sh, gsem):
    c = lax.axis_index("c")
    s = lax.axis_index("s")
    wid = c * NS + s

    pltpu.make_async_copy(sidx_hbm.at[wid], sidx_v, gsem).start()
    pltpu.make_async_copy(didx_hbm.at[wid], didx_v, gsem).start()
    pltpu.make_async_copy(b_hbm, bbuf, gsem).start()

    def fire_h_inputs(base, nrows, hb, tb, pb):
        pltpu.make_async_copy(parts_hbm.at[0, pl.ds(base, nrows)], hb, gsem).start()
        pltpu.make_async_copy(parts_hbm.at[1, pl.ds(base, nrows)], tb, gsem).start()
        pltpu.make_async_copy(p_hbm.at[pl.ds(base, nrows)], pb, gsem).start()

    def wait_h_inputs(base, nrows, hb, tb, pb):
        pltpu.make_async_copy(parts_hbm.at[0, pl.ds(base, nrows)], hb, gsem).wait()
        pltpu.make_async_copy(parts_hbm.at[1, pl.ds(base, nrows)], tb, gsem).wait()
        pltpu.make_async_copy(p_hbm.at[pl.ds(base, nrows)], pb, gsem).wait()

    def compute_h(base, nrows, hb, tb, pb):
        bvec = bbuf[0]

        def relu1(i, carry):
            hb[i] = jnp.maximum(hb[i] + tb[i] - pb[i] + bvec, 0.0)
            return carry
        lax.fori_loop(0, nrows, relu1, 0)
        pltpu.sync_copy(hb, stage_sh.at[pl.ds(base, nrows)])
        pltpu.sync_copy(hb, acc_sh.at[pl.ds(base, nrows)])

        @pl.when(c == 0)
        def _():
            pltpu.sync_copy(hb, h_hbm.at[pl.ds(base, nrows)])

    fire_h_inputs(s * RPT, RPT, hbuf, tbuf, pbuf)

    @pl.when(s == 0)
    def _():
        fire_h_inputs(REM_BASE, REM, hrem, trem, prem)

    pltpu.make_async_copy(sidx_hbm.at[wid], sidx_v, gsem).wait()
    pltpu.make_async_copy(didx_hbm.at[wid], didx_v, gsem).wait()
    pltpu.make_async_copy(b_hbm, bbuf, gsem).wait()
    wait_h_inputs(s * RPT, RPT, hbuf, tbuf, pbuf)
    compute_h(s * RPT, RPT, hbuf, tbuf, pbuf)

    @pl.when(s == 0)
    def _():
        wait_h_inputs(REM_BASE, REM, hrem, trem, prem)
        compute_h(REM_BASE, REM, hrem, trem, prem)
    plsc.subcore_barrier()

    def agg_pass(p, carry):
        for k in range(2):
            pltpu.make_async_copy(
                stage_sh.at[sidx_v.at[2 * p + k]], rows_v.at[k], gsem).start()
        for k in range(2):
            pltpu.make_async_copy(
                stage_sh.at[sidx_v.at[2 * p + k]], rows_v.at[k], gsem).wait()
        for k in range(2):
            pltpu.async_copy(
                rows_v.at[k], acc_sh.at[didx_v.at[2 * p + k]], gsem, add=True)
        for k in range(2):
            pltpu.make_async_copy(
                rows_v.at[k], acc_sh.at[didx_v.at[2 * p + k]], gsem).wait()
        return carry
    lax.fori_loop(0, CPT // 2, agg_pass, 0)
    plsc.subcore_barrier()

    pltpu.sync_copy(acc_sh.at[pl.ds(s * RPT, RPT)], out_hbm.at[c, pl.ds(s * RPT, RPT)])

    @pl.when(s == 0)
    def _():
        pltpu.sync_copy(acc_sh.at[pl.ds(REM_BASE, REM)], out_hbm.at[c, pl.ds(REM_BASE, REM)])


def _mm1_body(x_ref, w_ref, o_ref):
    o_ref[...] = jnp.dot(x_ref[...], w_ref[...],
                         preferred_element_type=jnp.float32)[:, :D_HID]


def _mm2_body(q_ref, h_ref, w_ref, b_ref, o_ref):
    agg = q_ref[0] + q_ref[1] - h_ref[...]
    o_ref[...] = jnp.dot(agg, w_ref[...], preferred_element_type=jnp.float32) + b_ref[...]


def kernel(features, edge_index, W1, b1, W2, b2):
    src = edge_index[0]
    dst = edge_index[1]
    pad = E_PAD - E
    src_p = jnp.concatenate([src, jnp.zeros((pad,), jnp.int32)]).reshape(NW, CPT, CHUNK)
    dst_p = jnp.concatenate([dst, jnp.full((pad,), TRASH, jnp.int32)]).reshape(NW, CPT, CHUNK)

    BM = 2000
    W1p = jnp.pad(W1, ((0, 0), (0, 128 - D_HID)))
    p1 = pl.pallas_call(
        _mm1_body,
        grid=(N // BM,),
        in_specs=[
            pl.BlockSpec((BM, D_IN), lambda i: (i, 0)),
            pl.BlockSpec((D_IN, 128), lambda i: (0, 0)),
        ],
        out_specs=pl.BlockSpec((BM, D_HID), lambda i: (i, 0)),
        out_shape=jax.ShapeDtypeStruct((N, D_HID), jnp.float32),
    )(features, W1p)

    parts1 = _sc_aggregate(p1, src_p, dst_p)

    BC = 2000
    h1, parts2 = _sc_combine_aggregate(parts1, p1, b1.reshape(1, D_HID), src_p, dst_p)

    W2p = jnp.pad(W2, ((0, 0), (0, 8 - D_OUT)))
    b2p = jnp.pad(b2, (0, 8 - D_OUT)).reshape(1, 8)
    out8 = pl.pallas_call(
        _mm2_body,
        grid=(N // BC,),
        in_specs=[
            pl.BlockSpec((NC, BC, D_HID), lambda i: (0, i, 0)),
            pl.BlockSpec((BC, D_HID), lambda i: (i, 0)),
            pl.BlockSpec((D_HID, 8), lambda i: (0, 0)),
            pl.BlockSpec((1, 8), lambda i: (0, 0)),
        ],
        out_specs=pl.BlockSpec((BC, 8), lambda i: (i, 0)),
        out_shape=jax.ShapeDtypeStruct((N, 8), jnp.float32),
    )(parts2, h1, W2p, b2p)

    return out8[:, :D_OUT]
